# baseline (speedup 1.0000x reference)
import numpy as np
import jax
import jax.numpy as jnp
from jax.experimental import pallas as pl
from jax.experimental.pallas import tpu as pltpu

_B, _T, _N, _F = 64, 50, 68, 128
_H = 256
_K = 5
_NP = _N + 2 * _K
_G = _B * _T
_GC = 64
_NCLS = 500
_ND = 2 * _K + 1


_NP2 = _N + 2 * _K
_TS = 5
_BC = 8
_TR = 10


def _deg_isqrt_np():
    deg = np.array([min(_N - 1, i + _K) - max(0, i - _K) + 1 for i in range(_N)],
                   np.float32)
    return (1.0 / np.sqrt(deg)).astype(np.float32)


_M = _TS * _BC


def _gcn_body(x_ref, ah_ref, w0, b0, w1, b1, w2, b2, out_ref):
    ah = ah_ref[...]
    h = jnp.reshape(jnp.transpose(x_ref[...], (1, 2, 0, 3)), (_N, _M, _F))
    for w_ref, b_ref in ((w0, b0), (w1, b1), (w2, b2)):
        hw = jnp.reshape(jnp.reshape(h, (_N * _M, _F)) @ w_ref[...],
                         (_N, _M, _F))
        agg = jax.lax.dot_general(ah, hw, (((1,), (0,)), ((), ())),
                                  preferred_element_type=jnp.float32)
        h = jnp.maximum(agg + b_ref[...][None], 0.0)
    emb = jnp.sum(h, axis=0) * (1.0 / _N)
    out_ref[...] = jnp.reshape(emb, (_TS, _BC, _F))


def _gcn(x4d, ah, w0, b0, w1, b1, w2, b2):
    const2 = lambda c, s: (0, 0)
    return pl.pallas_call(
        _gcn_body,
        grid=(_B // _BC, _T // _TS),
        in_specs=[
            pl.BlockSpec((_BC, _TS, _N, _F), lambda c, s: (c, s, 0, 0)),
            pl.BlockSpec((_N, _N), const2),
            pl.BlockSpec((_F, _F), const2), pl.BlockSpec((1, _F), const2),
            pl.BlockSpec((_F, _F), const2), pl.BlockSpec((1, _F), const2),
            pl.BlockSpec((_F, _F), const2), pl.BlockSpec((1, _F), const2),
        ],
        out_specs=pl.BlockSpec((_TS, _BC, _F), lambda c, s: (s, c, 0)),
        out_shape=jax.ShapeDtypeStruct((_T, _B, _F), jnp.float32),
    )(x4d, ah, w0, b0, w1, b1, w2, b2)


def _lstm_step(x, h_ref, c_ref, w_ref, b_ref):
    g = jnp.concatenate([x, h_ref[...]], axis=1) @ w_ref[...] + b_ref[...]
    i = jax.nn.sigmoid(g[:, 0:_H])
    f = jax.nn.sigmoid(g[:, _H:2 * _H])
    gg = jnp.tanh(g[:, 2 * _H:3 * _H])
    o = jax.nn.sigmoid(g[:, 3 * _H:4 * _H])
    c = f * c_ref[...] + i * gg
    h = o * jnp.tanh(c)
    c_ref[...] = c
    h_ref[...] = h
    return h


def _rec0_body(xf_ref, xb_ref, wf, bf, wb, bb, outf_ref, outb_ref,
               hf, cf, hb, cb):
    s = pl.program_id(0)

    @pl.when(s == 0)
    def _init():
        z = jnp.zeros((_B, _H), jnp.float32)
        hf[...] = z
        cf[...] = z
        hb[...] = z
        cb[...] = z

    for tt in range(_TR):
        outf_ref[tt] = _lstm_step(xf_ref[tt], hf, cf, wf, bf)
        outb_ref[_TR - 1 - tt] = _lstm_step(xb_ref[_TR - 1 - tt], hb, cb,
                                            wb, bb)


def _rec0(emb, wf, bf, wb, bb):
    const2 = lambda s: (0, 0)
    nsteps = _T // _TR
    return pl.pallas_call(
        _rec0_body,
        grid=(nsteps,),
        in_specs=[
            pl.BlockSpec((_TR, _B, _F), lambda s: (s, 0, 0)),
            pl.BlockSpec((_TR, _B, _F), lambda s: (nsteps - 1 - s, 0, 0)),
            pl.BlockSpec((_F + _H, 4 * _H), const2),
            pl.BlockSpec((1, 4 * _H), const2),
            pl.BlockSpec((_F + _H, 4 * _H), const2),
            pl.BlockSpec((1, 4 * _H), const2),
        ],
        out_specs=[
            pl.BlockSpec((_TR, _B, _H), lambda s: (s, 0, 0)),
            pl.BlockSpec((_TR, _B, _H), lambda s: (nsteps - 1 - s, 0, 0)),
        ],
        out_shape=[
            jax.ShapeDtypeStruct((_T, _B, _H), jnp.float32),
            jax.ShapeDtypeStruct((_T, _B, _H), jnp.float32),
        ],
        scratch_shapes=[pltpu.VMEM((_B, _H), jnp.float32)] * 4,
    )(emb, emb, wf, bf, wb, bb)


def _rec1_body(fa_ref, ba_ref, fd_ref, bd_ref, wf, bf, wb, bb,
               w1, b1, w2, b2, out_ref, hf, cf, hb, cb):
    s = pl.program_id(0)

    @pl.when(s == 0)
    def _init():
        z = jnp.zeros((_B, _H), jnp.float32)
        hf[...] = z
        cf[...] = z
        hb[...] = z
        cb[...] = z

    for tt in range(_TR):
        xf = jnp.concatenate([fa_ref[tt], ba_ref[tt]], axis=1)
        hfv = _lstm_step(xf, hf, cf, wf, bf)
        xb = jnp.concatenate([fd_ref[_TR - 1 - tt], bd_ref[_TR - 1 - tt]],
                             axis=1)
        hbv = _lstm_step(xb, hb, cb, wb, bb)

    @pl.when(s == _T // _TR - 1)
    def _cls():
        hcat = jnp.concatenate([hfv, hbv], axis=1)
        hid = jnp.maximum(hcat @ w1[...] + b1[...], 0.0)
        out_ref[...] = hid @ w2[...] + b2[...]


def _rec1(fw0, bw0, wf, bf, wb, bb, w1, b1, w2, b2):
    const2 = lambda s: (0, 0)
    nsteps = _T // _TR
    asc = lambda s: (s, 0, 0)
    dsc = lambda s: (nsteps - 1 - s, 0, 0)
    return pl.pallas_call(
        _rec1_body,
        grid=(nsteps,),
        in_specs=[
            pl.BlockSpec((_TR, _B, _H), asc),
            pl.BlockSpec((_TR, _B, _H), asc),
            pl.BlockSpec((_TR, _B, _H), dsc),
            pl.BlockSpec((_TR, _B, _H), dsc),
            pl.BlockSpec((2 * _H + _H, 4 * _H), const2),
            pl.BlockSpec((1, 4 * _H), const2),
            pl.BlockSpec((2 * _H + _H, 4 * _H), const2),
            pl.BlockSpec((1, 4 * _H), const2),
            pl.BlockSpec((2 * _H, _H), const2),
            pl.BlockSpec((1, _H), const2),
            pl.BlockSpec((_H, _NCLS), const2),
            pl.BlockSpec((1, _NCLS), const2),
        ],
        out_specs=pl.BlockSpec((_B, _NCLS), const2),
        out_shape=jax.ShapeDtypeStruct((_B, _NCLS), jnp.float32),
        scratch_shapes=[pltpu.VMEM((_B, _H), jnp.float32)] * 4,
    )(fw0, bw0, fw0, bw0, wf, bf, wb, bb, w1, b1, w2, b2)


def kernel(x_temporal, gcn_W0, gcn_b0, gcn_W1, gcn_b1, gcn_W2, gcn_b2,
           lstm_fw_Wih0, lstm_fw_Whh0, lstm_fw_b0,
           lstm_bw_Wih0, lstm_bw_Whh0, lstm_bw_b0,
           lstm_fw_Wih1, lstm_fw_Whh1, lstm_fw_b1,
           lstm_bw_Wih1, lstm_bw_Whh1, lstm_bw_b1,
           cls_W1, cls_b1, cls_W2, cls_b2):
    dis = _deg_isqrt_np()
    ah = np.zeros((_N, _N), np.float32)
    for i in range(_N):
        for j in range(max(0, i - _K), min(_N, i + _K + 1)):
            ah[i, j] = dis[i] * dis[j]
    ah = jnp.asarray(ah)

    emb = _gcn(x_temporal, ah, gcn_W0, gcn_b0[None], gcn_W1,
               gcn_b1[None], gcn_W2, gcn_b2[None])

    w0f = jnp.concatenate([lstm_fw_Wih0.T, lstm_fw_Whh0.T], axis=0)
    w0b = jnp.concatenate([lstm_bw_Wih0.T, lstm_bw_Whh0.T], axis=0)
    fw0, bw0 = _rec0(emb, w0f, lstm_fw_b0[None], w0b, lstm_bw_b0[None])

    w1f = jnp.concatenate([lstm_fw_Wih1.T, lstm_fw_Whh1.T], axis=0)
    w1b = jnp.concatenate([lstm_bw_Wih1.T, lstm_bw_Whh1.T], axis=0)
    return _rec1(fw0, bw0, w1f, lstm_fw_b1[None], w1b, lstm_bw_b1[None],
                 cls_W1, cls_b1[None], cls_W2, cls_b2[None])


# speedup vs baseline: 1.4117x; 1.4117x over previous
import numpy as np
import jax
import jax.numpy as jnp
from jax.experimental import pallas as pl
from jax.experimental.pallas import tpu as pltpu

_B, _T, _N, _F = 64, 50, 68, 128
_H = 256
_K = 5
_NP = _N + 2 * _K
_G = _B * _T
_GC = 64
_NCLS = 500
_ND = 2 * _K + 1


_NP2 = _N + 2 * _K
_TS = 5
_BC = 8
_TR = 10


def _deg_isqrt_np():
    deg = np.array([min(_N - 1, i + _K) - max(0, i - _K) + 1 for i in range(_N)],
                   np.float32)
    return (1.0 / np.sqrt(deg)).astype(np.float32)


def _gcn_body(x_ref, disj_ref, disi_ref, w0, b0, w1, b1, w2, b2, out_ref):
    zpad = jnp.zeros((_TS, _K, _BC, _F), jnp.float32)
    disj = disj_ref[...][None, :, None]
    disi = disi_ref[...][None, :, None]
    h = jnp.transpose(x_ref[...], (1, 2, 0, 3))
    for w_ref, b_ref in ((w0, b0), (w1, b1), (w2, b2)):
        hw = jnp.reshape(jnp.reshape(h, (_TS * _N * _BC, _F)) @ w_ref[...],
                         (_TS, _N, _BC, _F))
        p = jnp.concatenate([zpad, hw * disj, zpad], axis=1)
        p2 = p[:, 0:_NP2 - 1] + p[:, 1:_NP2]
        p4 = p2[:, 0:_NP2 - 3] + p2[:, 2:_NP2 - 1]
        p8 = p4[:, 0:_N] + p4[:, 4:_N + 4]
        win = p8 + p2[:, 8:_N + 8] + p[:, 10:_N + 10]
        h = jnp.maximum(win * disi + b_ref[...][None, None], 0.0)
    emb = jnp.sum(h, axis=1) * (1.0 / _N)
    out_ref[...] = emb


def _gcn(x4d, disj, disi, w0, b0, w1, b1, w2, b2):
    const2 = lambda c, s: (0, 0)
    return pl.pallas_call(
        _gcn_body,
        grid=(_B // _BC, _T // _TS),
        in_specs=[
            pl.BlockSpec((_BC, _TS, _N, _F), lambda c, s: (c, s, 0, 0)),
            pl.BlockSpec((_N, _F), const2),
            pl.BlockSpec((_N, _F), const2),
            pl.BlockSpec((_F, _F), const2), pl.BlockSpec((1, _F), const2),
            pl.BlockSpec((_F, _F), const2), pl.BlockSpec((1, _F), const2),
            pl.BlockSpec((_F, _F), const2), pl.BlockSpec((1, _F), const2),
        ],
        out_specs=pl.BlockSpec((_TS, _BC, _F), lambda c, s: (s, c, 0)),
        out_shape=jax.ShapeDtypeStruct((_T, _B, _F), jnp.float32),
    )(x4d, disj, disi, w0, b0, w1, b1, w2, b2)


def _lstm_step(x, h, c, w_ref, b_ref):
    g = jnp.concatenate([x, h], axis=1) @ w_ref[...] + b_ref[...]
    i = jax.nn.sigmoid(g[:, 0:_H])
    f = jax.nn.sigmoid(g[:, _H:2 * _H])
    gg = jnp.tanh(g[:, 2 * _H:3 * _H])
    o = jax.nn.sigmoid(g[:, 3 * _H:4 * _H])
    c = f * c + i * gg
    return o * jnp.tanh(c), c


def _lstm_body(emb_ref, w0f, b0f, w0b, b0b, w1f, b1f, w1b, b1b,
               w1, b1, w2, b2, out_ref, f0_buf, b0_buf):
    z = jnp.zeros((_B, _H), jnp.float32)
    hf, cf, hb, cb = z, z, z, z
    for t in range(_T):
        hf, cf = _lstm_step(emb_ref[t], hf, cf, w0f, b0f)
        f0_buf[t] = hf
        hb, cb = _lstm_step(emb_ref[_T - 1 - t], hb, cb, w0b, b0b)
        b0_buf[_T - 1 - t] = hb
    hf, cf, hb, cb = z, z, z, z
    for t in range(_T):
        xf = jnp.concatenate([f0_buf[t], b0_buf[t]], axis=1)
        hf, cf = _lstm_step(xf, hf, cf, w1f, b1f)
        xb = jnp.concatenate([f0_buf[_T - 1 - t], b0_buf[_T - 1 - t]], axis=1)
        hb, cb = _lstm_step(xb, hb, cb, w1b, b1b)
    hcat = jnp.concatenate([hf, hb], axis=1)
    hid = jnp.maximum(hcat @ w1[...] + b1[...], 0.0)
    out_ref[...] = hid @ w2[...] + b2[...]


def _lstm(emb, w0f, b0f, w0b, b0b, w1f, b1f, w1b, b1b, w1, b1, w2, b2):
    const2 = lambda: (0, 0)
    const3 = lambda: (0, 0, 0)
    return pl.pallas_call(
        _lstm_body,
        grid=(),
        in_specs=[
            pl.BlockSpec((_T, _B, _F), const3),
            pl.BlockSpec((_F + _H, 4 * _H), const2),
            pl.BlockSpec((1, 4 * _H), const2),
            pl.BlockSpec((_F + _H, 4 * _H), const2),
            pl.BlockSpec((1, 4 * _H), const2),
            pl.BlockSpec((3 * _H, 4 * _H), const2),
            pl.BlockSpec((1, 4 * _H), const2),
            pl.BlockSpec((3 * _H, 4 * _H), const2),
            pl.BlockSpec((1, 4 * _H), const2),
            pl.BlockSpec((2 * _H, _H), const2),
            pl.BlockSpec((1, _H), const2),
            pl.BlockSpec((_H, _NCLS), const2),
            pl.BlockSpec((1, _NCLS), const2),
        ],
        out_specs=pl.BlockSpec((_B, _NCLS), const2),
        out_shape=jax.ShapeDtypeStruct((_B, _NCLS), jnp.float32),
        scratch_shapes=[pltpu.VMEM((_T, _B, _H), jnp.float32)] * 2,
    )(emb, w0f, b0f, w0b, b0b, w1f, b1f, w1b, b1b, w1, b1, w2, b2)


def kernel(x_temporal, gcn_W0, gcn_b0, gcn_W1, gcn_b1, gcn_W2, gcn_b2,
           lstm_fw_Wih0, lstm_fw_Whh0, lstm_fw_b0,
           lstm_bw_Wih0, lstm_bw_Whh0, lstm_bw_b0,
           lstm_fw_Wih1, lstm_fw_Whh1, lstm_fw_b1,
           lstm_bw_Wih1, lstm_bw_Whh1, lstm_bw_b1,
           cls_W1, cls_b1, cls_W2, cls_b2):
    dis = np.repeat(_deg_isqrt_np()[:, None], _F, axis=1)
    disj = jnp.asarray(dis)
    disi = jnp.asarray(dis)

    emb = _gcn(x_temporal, disj, disi, gcn_W0, gcn_b0[None], gcn_W1,
               gcn_b1[None], gcn_W2, gcn_b2[None])

    w0f = jnp.concatenate([lstm_fw_Wih0.T, lstm_fw_Whh0.T], axis=0)
    w0b = jnp.concatenate([lstm_bw_Wih0.T, lstm_bw_Whh0.T], axis=0)
    w1f = jnp.concatenate([lstm_fw_Wih1.T, lstm_fw_Whh1.T], axis=0)
    w1b = jnp.concatenate([lstm_bw_Wih1.T, lstm_bw_Whh1.T], axis=0)
    return _lstm(emb, w0f, lstm_fw_b0[None], w0b, lstm_bw_b0[None],
                 w1f, lstm_fw_b1[None], w1b, lstm_bw_b1[None],
                 cls_W1, cls_b1[None], cls_W2, cls_b2[None])
